# baseline (speedup 1.0000x reference)
import jax
import jax.numpy as jnp
from jax import lax
from jax.experimental import pallas as pl
from jax.experimental.pallas import tpu as pltpu
from jax.experimental.pallas import tpu_sc as plsc

B, F, D = 64, 2048, 128
D3 = 3 * D
NV = D // 16
CH = 256
NW = 32
REP = B // NW
BIG = 3e38

_DNUMS = lax.GatherDimensionNumbers(
    offset_dims=(), collapsed_slice_dims=(0,), start_index_map=(0,))


def _lanes_fold(x, op):
    for sh in (1, 2, 4, 8):
        idx = jnp.bitwise_xor(lax.iota(jnp.int32, 16), sh)
        perm = lax.gather(x, idx[:, None], _DNUMS, (1,),
                          mode=lax.GatherScatterMode.PROMISE_IN_BOUNDS)
        x = op(x, perm)
    return x


def _sc_body(nb_hbm, q_hbm, fr_hbm, fa1_hbm, fa2_hbm, out_hbm,
             nb_v, q_v, fr_v, fa1_v, fa2_v, res_v):
    wid = lax.axis_index("s") * 2 + lax.axis_index("c")
    pltpu.sync_copy(nb_hbm, nb_v.at[pl.ds(0, B)])

    for rep in range(REP):
        b = wid + rep * NW
        pltpu.sync_copy(q_hbm.at[b], q_v)
        n = nb_v[pl.ds(b, 16)][0]
        trips = (n + CH - 1) // CH

        q2 = [q_v[pl.ds(16 * i, 16)] * 2.0 for i in range(3 * NV)]
        nqv = jnp.zeros((16,), jnp.float32)
        for i in range(3 * NV):
            qv = q_v[pl.ds(16 * i, 16)]
            nqv = nqv + qv * qv
        nqs = _lanes_fold(nqv, jnp.add)

        def chunk_body(c, smin):
            base = c * CH
            pltpu.sync_copy(fr_hbm.at[b, pl.ds(base, CH)], fr_v)
            pltpu.sync_copy(fa1_hbm.at[b, pl.ds(base, CH)], fa1_v)
            pltpu.sync_copy(fa2_hbm.at[b, pl.ds(base, CH)], fa2_v)

            def fact_body(f, smin2):
                acc = jnp.zeros((16,), jnp.float32)
                bufs = (fr_v, fa1_v, fa2_v)
                for a in range(3):
                    for i in range(NV):
                        v = bufs[a][f, pl.ds(16 * i, 16)]
                        acc = acc + v * (v - q2[a * NV + i])
                s = _lanes_fold(acc, jnp.add)
                invalid = (base + f >= n).astype(jnp.float32)
                s = s + invalid * BIG
                return jnp.minimum(smin2, s)

            return lax.fori_loop(0, CH, fact_body, smin)

        smin = lax.fori_loop(0, trips, chunk_body,
                             jnp.full((16,), BIG, jnp.float32))
        res_v[...] = jnp.exp(-0.5 * (smin + nqs))
        pltpu.sync_copy(res_v, out_hbm.at[b])


def kernel(rel, arg1, arg2, facts_rel, facts_arg1, facts_arg2, nb_facts):
    mesh = plsc.VectorSubcoreMesh(core_axis_name="c", subcore_axis_name="s")
    qcat = jnp.concatenate([rel, arg1, arg2], axis=1)

    f = pl.kernel(
        _sc_body,
        mesh=mesh,
        out_type=jax.ShapeDtypeStruct((B, 16), jnp.float32),
        scratch_types=[
            pltpu.VMEM((B + 16,), jnp.int32),
            pltpu.VMEM((D3,), jnp.float32),
            pltpu.VMEM((CH, D), jnp.float32),
            pltpu.VMEM((CH, D), jnp.float32),
            pltpu.VMEM((CH, D), jnp.float32),
            pltpu.VMEM((16,), jnp.float32),
        ],
    )
    out = f(nb_facts, qcat, facts_rel, facts_arg1, facts_arg2)
    return out[:, 0]


# speedup vs baseline: 1.9683x; 1.9683x over previous
import jax
import jax.numpy as jnp
from jax import lax
from jax.experimental import pallas as pl
from jax.experimental.pallas import tpu as pltpu
from jax.experimental.pallas import tpu_sc as plsc

B, F, D = 64, 2048, 128
D3 = 3 * D
NV = D // 16
CH = 128
NW = 32
REP = B // NW
BIG = 3e38

_DNUMS = lax.GatherDimensionNumbers(
    offset_dims=(), collapsed_slice_dims=(0,), start_index_map=(0,))


def _lanes_fold(x, op):
    for sh in (1, 2, 4, 8):
        idx = jnp.bitwise_xor(lax.iota(jnp.int32, 16), sh)
        perm = lax.gather(x, idx[:, None], _DNUMS, (1,),
                          mode=lax.GatherScatterMode.PROMISE_IN_BOUNDS)
        x = op(x, perm)
    return x


def _sc_body(nb_hbm, q_hbm, fr_hbm, fa1_hbm, fa2_hbm, out_hbm,
             nb_v, q_v, fr_v, fa1_v, fa2_v, res_v, sems):
    wid = lax.axis_index("s") * 2 + lax.axis_index("c")
    pltpu.sync_copy(nb_hbm, nb_v.at[pl.ds(0, B)])

    for rep in range(REP):
        b = wid + rep * NW
        pltpu.sync_copy(q_hbm.at[b], q_v)
        n = nb_v[pl.ds(b, 16)][0]
        trips = (n + CH - 1) // CH

        q2 = [q_v[pl.ds(16 * i, 16)] * 2.0 for i in range(3 * NV)]
        nqv = jnp.zeros((16,), jnp.float32)
        for i in range(3 * NV):
            qv = q_v[pl.ds(16 * i, 16)]
            nqv = nqv + qv * qv
        nqs = _lanes_fold(nqv, jnp.add)

        def copies(c, slot):
            src = (fr_hbm, fa1_hbm, fa2_hbm)
            dst = (fr_v, fa1_v, fa2_v)
            return [
                pltpu.make_async_copy(
                    src[i].at[b, pl.ds(c * CH, CH)], dst[i].at[slot],
                    sems.at[i, slot])
                for i in range(3)
            ]

        def start(c, slot):
            for cp in copies(c, slot):
                cp.start()

        def wait(c, slot):
            for cp in copies(c, slot):
                cp.wait()

        start(0, 0)

        def chunk_body(c, smin):
            base = c * CH
            slot = jnp.bitwise_and(c, 1)

            @pl.when(c + 1 < trips)
            def _():
                start(c + 1, jnp.bitwise_and(c + 1, 1))

            wait(c, slot)

            def fact_body(f, smin2):
                acc = jnp.zeros((16,), jnp.float32)
                bufs = (fr_v, fa1_v, fa2_v)
                for a in range(3):
                    for i in range(NV):
                        v = bufs[a][slot, f, pl.ds(16 * i, 16)]
                        acc = acc + v * (v - q2[a * NV + i])
                s = _lanes_fold(acc, jnp.add)
                invalid = (base + f >= n).astype(jnp.float32)
                s = s + invalid * BIG
                return jnp.minimum(smin2, s)

            return lax.fori_loop(0, CH, fact_body, smin)

        smin = lax.fori_loop(0, trips, chunk_body,
                             jnp.full((16,), BIG, jnp.float32))
        res_v[...] = jnp.exp(-0.5 * (smin + nqs))
        pltpu.sync_copy(res_v, out_hbm.at[b])


def kernel(rel, arg1, arg2, facts_rel, facts_arg1, facts_arg2, nb_facts):
    mesh = plsc.VectorSubcoreMesh(core_axis_name="c", subcore_axis_name="s")
    qcat = jnp.concatenate([rel, arg1, arg2], axis=1)

    f = pl.kernel(
        _sc_body,
        mesh=mesh,
        out_type=jax.ShapeDtypeStruct((B, 16), jnp.float32),
        scratch_types=[
            pltpu.VMEM((B + 16,), jnp.int32),
            pltpu.VMEM((D3,), jnp.float32),
            pltpu.VMEM((2, CH, D), jnp.float32),
            pltpu.VMEM((2, CH, D), jnp.float32),
            pltpu.VMEM((2, CH, D), jnp.float32),
            pltpu.VMEM((16,), jnp.float32),
            pltpu.SemaphoreType.DMA((3, 2)),
        ],
    )
    out = f(nb_facts, qcat, facts_rel, facts_arg1, facts_arg2)
    return out[:, 0]
